# baseline (device time: 26528 ns/iter reference)
import jax
import jax.numpy as jnp
from jax import lax
from jax.experimental import pallas as pl
from jax.experimental.pallas import tpu as pltpu

N_DEV = 4
N_LAYERS = 3
CH = 2
SEND_ORDER = (2, 1, 3)
RECV_ORDER = (1, 3, 2)


def kernel(x, Win0, Wout0, Win1, Wout1, Win2, Wout2):
    b, d_in = x.shape
    h_dim = Win0.shape[1]
    d_out = Wout0.shape[1]
    cw = h_dim // CH

    def body(x_ref, wins_ref, wouts_ref, out_ref, send_buf, comm_ref,
             send_sems, recv_sems):
        my = lax.axis_index("i")

        barrier_sem = pltpu.get_barrier_semaphore()
        for off in range(1, N_DEV):
            pl.semaphore_signal(
                barrier_sem, inc=1,
                device_id=((my + off) % N_DEV,),
                device_id_type=pl.DeviceIdType.MESH,
            )

        xv = x_ref[...].astype(jnp.bfloat16)
        for layer in range(N_LAYERS):
            rdmas = []
            pbs = []
            for c in range(CH):
                pc = jnp.dot(
                    xv, wins_ref[layer, :, c * cw:(c + 1) * cw],
                    preferred_element_type=jnp.float32,
                )
                pcb = pc.astype(jnp.bfloat16)
                send_buf[layer, c] = pcb
                if layer == 0 and c == 0:
                    pl.semaphore_wait(barrier_sem, N_DEV - 1)
                crs = {}
                for off in SEND_ORDER:
                    rdma = pltpu.make_async_remote_copy(
                        src_ref=send_buf.at[layer, c],
                        dst_ref=comm_ref.at[layer, c, off - 1],
                        send_sem=send_sems.at[layer, c, off - 1],
                        recv_sem=recv_sems.at[layer, c, off - 1],
                        device_id=((my + off) % N_DEV,),
                        device_id_type=pl.DeviceIdType.MESH,
                    )
                    rdma.start()
                    crs[off] = rdma
                rdmas.append(crs)
                pbs.append(pcb)

            nxt = None
            for c in range(CH):
                acc = pbs[c]
                for off in RECV_ORDER:
                    rdmas[c][off].wait_recv()
                    acc = acc + comm_ref[layer, c, off - 1]
                hc = jnp.maximum(acc, jnp.bfloat16(0.0))
                contrib = jnp.dot(
                    hc, wouts_ref[layer, c * cw:(c + 1) * cw, :],
                    preferred_element_type=jnp.float32,
                )
                nxt = contrib if nxt is None else nxt + contrib

            for crs in rdmas:
                for off in SEND_ORDER:
                    crs[off].wait_send()

            if layer == N_LAYERS - 1:
                out_ref[...] = nxt.astype(jnp.bfloat16)
            else:
                xv = nxt.astype(jnp.bfloat16)

    wins = jnp.stack([Win0, Win1, Win2]).astype(jnp.bfloat16)
    wouts = jnp.stack([Wout0, Wout1, Wout2]).astype(jnp.bfloat16)
    return pl.pallas_call(
        body,
        out_shape=jax.ShapeDtypeStruct((b, d_out), jnp.bfloat16),
        in_specs=[pl.BlockSpec(memory_space=pltpu.VMEM)] * 3,
        out_specs=pl.BlockSpec(memory_space=pltpu.VMEM),
        scratch_shapes=[
            pltpu.VMEM((N_LAYERS, CH, b, cw), jnp.bfloat16),
            pltpu.VMEM((N_LAYERS, CH, N_DEV - 1, b, cw), jnp.bfloat16),
            pltpu.SemaphoreType.DMA((N_LAYERS, CH, N_DEV - 1)),
            pltpu.SemaphoreType.DMA((N_LAYERS, CH, N_DEV - 1)),
        ],
        compiler_params=pltpu.CompilerParams(collective_id=0),
    )(x, wins, wouts)


# device time: 26386 ns/iter; 1.0054x vs baseline; 1.0054x over previous
import jax
import jax.numpy as jnp
from jax import lax
from jax.experimental import pallas as pl
from jax.experimental.pallas import tpu as pltpu

N_DEV = 4
N_LAYERS = 3
SEND_ORDER = (2, 1, 3)
RECV_ORDER = (1, 3, 2)


def kernel(x, Win0, Wout0, Win1, Wout1, Win2, Wout2):
    b, d_in = x.shape
    h_dim = Win0.shape[1]
    d_out = Wout0.shape[1]

    def body(x_ref, wins_ref, wouts_ref, out_ref, send_buf, comm_ref,
             send_sems, recv_sems):
        my = lax.axis_index("i")

        barrier_sem = pltpu.get_barrier_semaphore()
        for off in range(1, N_DEV):
            pl.semaphore_signal(
                barrier_sem, inc=1,
                device_id=((my + off) % N_DEV,),
                device_id_type=pl.DeviceIdType.MESH,
            )

        xv = x_ref[...].astype(jnp.bfloat16)
        for layer in range(N_LAYERS):
            partial = jnp.dot(
                xv, wins_ref[layer], preferred_element_type=jnp.float32,
            )
            pb = partial.astype(jnp.bfloat16)
            send_buf[layer] = pb
            if layer == 0:
                pl.semaphore_wait(barrier_sem, N_DEV - 1)

            rdmas = {}
            for off in SEND_ORDER:
                rdma = pltpu.make_async_remote_copy(
                    src_ref=send_buf.at[layer],
                    dst_ref=comm_ref.at[layer, off - 1],
                    send_sem=send_sems.at[layer, off - 1],
                    recv_sem=recv_sems.at[layer, off - 1],
                    device_id=((my + off) % N_DEV,),
                    device_id_type=pl.DeviceIdType.MESH,
                )
                rdma.start()
                rdmas[off] = rdma

            acc = pb
            for off in RECV_ORDER:
                rdmas[off].wait_recv()
                acc = acc + comm_ref[layer, off - 1]
            for off in SEND_ORDER:
                rdmas[off].wait_send()

            h = jnp.maximum(acc, jnp.bfloat16(0.0))
            nxt = jnp.dot(
                h, wouts_ref[layer], preferred_element_type=jnp.float32,
            )
            if layer == N_LAYERS - 1:
                out_ref[...] = nxt.astype(jnp.bfloat16)
            else:
                xv = nxt.astype(jnp.bfloat16)

    wins = jnp.stack([Win0, Win1, Win2]).astype(jnp.bfloat16)
    wouts = jnp.stack([Wout0, Wout1, Wout2]).astype(jnp.bfloat16)
    return pl.pallas_call(
        body,
        out_shape=jax.ShapeDtypeStruct((b, d_out), jnp.bfloat16),
        in_specs=[pl.BlockSpec(memory_space=pltpu.VMEM)] * 3,
        out_specs=pl.BlockSpec(memory_space=pltpu.VMEM),
        scratch_shapes=[
            pltpu.VMEM((N_LAYERS, b, h_dim), jnp.bfloat16),
            pltpu.VMEM((N_LAYERS, N_DEV - 1, b, h_dim), jnp.bfloat16),
            pltpu.SemaphoreType.DMA((N_LAYERS, N_DEV - 1)),
            pltpu.SemaphoreType.DMA((N_LAYERS, N_DEV - 1)),
        ],
        compiler_params=pltpu.CompilerParams(collective_id=0),
    )(x, wins, wouts)
